# baseline (speedup 1.0000x reference)
import functools

import jax
import jax.numpy as jnp
from jax.experimental import pallas as pl
from jax.experimental.pallas import tpu as pltpu

ALPHA = 0.2


def _pick_block(n: int, target: int) -> int:
    b = min(target, n)
    b -= b % 8
    while b >= 8:
        if n % b == 0:
            return b
        b -= 8
    return n


def _prologue_body(x_ref, w_ref, a1_ref, a2_ref, wh_ref, c_ref, r_ref):
    wh = jnp.dot(x_ref[...], w_ref[...], preferred_element_type=jnp.float32)
    f_out = wh.shape[1]
    s_dst = jnp.dot(wh, a2_ref[...], preferred_element_type=jnp.float32)
    s_src = jnp.dot(wh, a1_ref[...], preferred_element_type=jnp.float32)
    v2 = jnp.exp(-ALPHA * s_dst)
    c_ref[...] = jnp.exp(-(1.0 - ALPHA) * s_dst)
    r_ref[...] = jnp.exp((1.0 - ALPHA) * s_src)
    lane = jax.lax.broadcasted_iota(jnp.int32, (wh.shape[0], 8), 1)
    wh_ref[:, :f_out] = v2 * wh
    wh_ref[:, f_out:] = jnp.where(lane == 0, v2, 0.0)


def _main_body(nsteps, nbuf, bm, f_out, adj_hbm, wh_ref, c_ref, r_ref,
               out_ref, buf_ref, sem):
    k = pl.program_id(0)

    @pl.when(k == 0)
    def _warmup():
        for b in range(nbuf):
            pltpu.make_async_copy(
                adj_hbm.at[pl.ds(b * bm, bm), :], buf_ref.at[b],
                sem.at[b]).start()

    slot = jax.lax.rem(k, nbuf)
    pltpu.make_async_copy(
        adj_hbm.at[pl.ds(k * bm, bm), :], buf_ref.at[slot],
        sem.at[slot]).wait()

    e = buf_ref[slot] * jnp.minimum(c_ref[...], r_ref[...])
    acc = jnp.dot(e, wh_ref[...], preferred_element_type=jnp.float32)
    h = acc[:, :f_out] / acc[:, f_out:f_out + 1]
    out_ref[...] = jnp.where(h > 0, h, jnp.exp(jnp.minimum(h, 0.0)) - 1.0)

    nxt = k + nbuf

    @pl.when(nxt < nsteps)
    def _prefetch():
        pltpu.make_async_copy(
            adj_hbm.at[pl.ds(nxt * bm, bm), :], buf_ref.at[slot],
            sem.at[slot]).start()


def kernel(input, adj, W, a):
    n, f_in = input.shape
    f_out = W.shape[1]
    a1 = a[0, :f_out].reshape(f_out, 1)
    a2 = a[0, f_out:].reshape(f_out, 1)

    bp = _pick_block(n, 2000)
    np_ = n // bp
    wh, cc, rc = pl.pallas_call(
        _prologue_body,
        grid=(np_,),
        in_specs=[
            pl.BlockSpec((bp, f_in), lambda i: (i, 0)),
            pl.BlockSpec((f_in, f_out), lambda i: (0, 0)),
            pl.BlockSpec((f_out, 1), lambda i: (0, 0)),
            pl.BlockSpec((f_out, 1), lambda i: (0, 0)),
        ],
        out_specs=[
            pl.BlockSpec((bp, f_out + 8), lambda i: (i, 0)),
            pl.BlockSpec((bp, 1), lambda i: (i, 0)),
            pl.BlockSpec((bp, 1), lambda i: (i, 0)),
        ],
        out_shape=[
            jax.ShapeDtypeStruct((n, f_out + 8), jnp.float32),
            jax.ShapeDtypeStruct((n, 1), jnp.float32),
            jax.ShapeDtypeStruct((n, 1), jnp.float32),
        ],
    )(input, W, a1, a2)

    c = cc.reshape(1, n)

    bm = _pick_block(n, 200)
    ni = n // bm
    nbuf = min(6, ni)
    out = pl.pallas_call(
        functools.partial(_main_body, ni, nbuf, bm, f_out),
        grid=(ni,),
        in_specs=[
            pl.BlockSpec(memory_space=pltpu.MemorySpace.HBM),
            pl.BlockSpec((n, f_out + 8), lambda k: (0, 0)),
            pl.BlockSpec((1, n), lambda k: (0, 0)),
            pl.BlockSpec((bm, 1), lambda k: (k, 0)),
        ],
        out_specs=pl.BlockSpec((bm, f_out), lambda k: (k, 0)),
        out_shape=jax.ShapeDtypeStruct((n, f_out), jnp.float32),
        scratch_shapes=[
            pltpu.VMEM((nbuf, bm, n), jnp.float32),
            pltpu.SemaphoreType.DMA((nbuf,)),
        ],
        compiler_params=pltpu.CompilerParams(
            dimension_semantics=("arbitrary",),
            vmem_limit_bytes=100 * 1024 * 1024),
    )(adj, wh, c, rc)
    return out


# speedup vs baseline: 1.0513x; 1.0513x over previous
import functools

import jax
import jax.numpy as jnp
from jax.experimental import pallas as pl
from jax.experimental.pallas import tpu as pltpu

ALPHA = 0.2


def _pick_block(n: int, target: int) -> int:
    b = min(target, n)
    b -= b % 8
    while b >= 8:
        if n % b == 0:
            return b
        b -= 8
    return n


def _prologue_body(x_ref, w_ref, a1_ref, a2_ref, wh_ref, c_ref, r_ref):
    wh = jnp.dot(x_ref[...], w_ref[...], preferred_element_type=jnp.float32)
    f_out = wh.shape[1]
    s_dst = jnp.dot(wh, a2_ref[...], preferred_element_type=jnp.float32)
    s_src = jnp.dot(wh, a1_ref[...], preferred_element_type=jnp.float32)
    v2 = jnp.exp(-ALPHA * s_dst)
    c_ref[...] = jnp.exp(-(1.0 - ALPHA) * s_dst)
    r_ref[...] = jnp.exp((1.0 - ALPHA) * s_src)
    lane = jax.lax.broadcasted_iota(jnp.int32, (wh.shape[0], 8), 1)
    wh_ref[:, :f_out] = v2 * wh
    wh_ref[:, f_out:] = jnp.where(lane == 0, v2, 0.0)


def _main_body(f_out, adj_ref, wh_ref, c_ref, r_ref, out_ref):
    e = adj_ref[...] * jnp.minimum(c_ref[...], r_ref[...])
    acc = jnp.dot(e, wh_ref[...], preferred_element_type=jnp.float32)
    h = acc[:, :f_out] / acc[:, f_out:f_out + 1]
    out_ref[...] = jnp.where(h > 0, h, jnp.exp(jnp.minimum(h, 0.0)) - 1.0)


def kernel(input, adj, W, a):
    n, f_in = input.shape
    f_out = W.shape[1]
    a1 = a[0, :f_out].reshape(f_out, 1)
    a2 = a[0, f_out:].reshape(f_out, 1)

    bp = _pick_block(n, 2000)
    np_ = n // bp
    wh, cc, rc = pl.pallas_call(
        _prologue_body,
        grid=(np_,),
        in_specs=[
            pl.BlockSpec((bp, f_in), lambda i: (i, 0)),
            pl.BlockSpec((f_in, f_out), lambda i: (0, 0)),
            pl.BlockSpec((f_out, 1), lambda i: (0, 0)),
            pl.BlockSpec((f_out, 1), lambda i: (0, 0)),
        ],
        out_specs=[
            pl.BlockSpec((bp, f_out + 8), lambda i: (i, 0)),
            pl.BlockSpec((bp, 1), lambda i: (i, 0)),
            pl.BlockSpec((bp, 1), lambda i: (i, 0)),
        ],
        out_shape=[
            jax.ShapeDtypeStruct((n, f_out + 8), jnp.float32),
            jax.ShapeDtypeStruct((n, 1), jnp.float32),
            jax.ShapeDtypeStruct((n, 1), jnp.float32),
        ],
    )(input, W, a1, a2)

    c = cc.reshape(1, n)

    bm = _pick_block(n, 400)
    ni = n // bm
    out = pl.pallas_call(
        functools.partial(_main_body, f_out),
        grid=(ni,),
        in_specs=[
            pl.BlockSpec((bm, n), lambda k: (k, 0)),
            pl.BlockSpec((n, f_out + 8), lambda k: (0, 0)),
            pl.BlockSpec((1, n), lambda k: (0, 0)),
            pl.BlockSpec((bm, 1), lambda k: (k, 0)),
        ],
        out_specs=pl.BlockSpec((bm, f_out), lambda k: (k, 0)),
        out_shape=jax.ShapeDtypeStruct((n, f_out), jnp.float32),
        compiler_params=pltpu.CompilerParams(
            dimension_semantics=("arbitrary",)),
    )(adj, wh, c, rc)
    return out


# speedup vs baseline: 1.1794x; 1.1218x over previous
import functools

import jax
import jax.numpy as jnp
from jax.experimental import pallas as pl
from jax.experimental.pallas import tpu as pltpu

ALPHA = 0.2


def _pick_block(n: int, target: int) -> int:
    b = min(target, n)
    b -= b % 8
    while b >= 8:
        if n % b == 0:
            return b
        b -= 8
    return n


def _body(bm, f_out, adj_ref, x_ref, w_ref, a1_ref, a2_ref, out_ref,
          wh_ref, c_ref, r_ref):
    k = pl.program_id(0)

    @pl.when(k == 0)
    def _prologue():
        wh = jnp.dot(x_ref[...], w_ref[...], preferred_element_type=jnp.float32)
        s_dst = jnp.dot(wh, a2_ref[...], preferred_element_type=jnp.float32)
        s_src = jnp.dot(wh, a1_ref[...], preferred_element_type=jnp.float32)
        v2 = jnp.exp(-ALPHA * s_dst)
        c_ref[...] = jnp.exp(-(1.0 - ALPHA) * s_dst).T
        r_ref[...] = jnp.exp((1.0 - ALPHA) * s_src)
        lane = jax.lax.broadcasted_iota(jnp.int32, (wh.shape[0], 8), 1)
        wh_ref[:, :f_out] = v2 * wh
        wh_ref[:, f_out:] = jnp.where(lane == 0, v2, 0.0)

    r_k = r_ref[pl.ds(k * bm, bm), :]
    e = adj_ref[...] * jnp.minimum(c_ref[...], r_k)
    acc = jnp.dot(e, wh_ref[...], preferred_element_type=jnp.float32)
    h = acc[:, :f_out] / acc[:, f_out:f_out + 1]
    out_ref[...] = jnp.where(h > 0, h, jnp.exp(jnp.minimum(h, 0.0)) - 1.0)


def kernel(input, adj, W, a):
    n, f_in = input.shape
    f_out = W.shape[1]
    a1 = a[0, :f_out].reshape(f_out, 1)
    a2 = a[0, f_out:].reshape(f_out, 1)

    bm = _pick_block(n, 400)
    ni = n // bm
    out = pl.pallas_call(
        functools.partial(_body, bm, f_out),
        grid=(ni,),
        in_specs=[
            pl.BlockSpec((bm, n), lambda k: (k, 0)),
            pl.BlockSpec((n, f_in), lambda k: (0, 0)),
            pl.BlockSpec((f_in, f_out), lambda k: (0, 0)),
            pl.BlockSpec((f_out, 1), lambda k: (0, 0)),
            pl.BlockSpec((f_out, 1), lambda k: (0, 0)),
        ],
        out_specs=pl.BlockSpec((bm, f_out), lambda k: (k, 0)),
        out_shape=jax.ShapeDtypeStruct((n, f_out), jnp.float32),
        scratch_shapes=[
            pltpu.VMEM((n, f_out + 8), jnp.float32),
            pltpu.VMEM((1, n), jnp.float32),
            pltpu.VMEM((n, 1), jnp.float32),
        ],
        compiler_params=pltpu.CompilerParams(
            dimension_semantics=("arbitrary",),
            vmem_limit_bytes=100 * 1024 * 1024),
    )(adj, input, W, a1, a2)
    return out
